# baseline (speedup 1.0000x reference)
import functools

import jax
import jax.numpy as jnp
from jax import lax
from jax.experimental import pallas as pl
from jax.experimental.pallas import tpu as pltpu
from jax.experimental.pallas import tpu_sc as plsc

SEQ = 50
HALF_SEQ = 25
LANES = 16
BCHUNK = 128
DGROUP = 8


def _make_sc_gather(Bq, V, D):
    info = plsc.get_sparse_core_info()
    num_workers = info.num_cores * info.num_subcores
    n_dgroups = D // DGROUP
    n_quarters = num_workers // n_dgroups
    b_per_w = Bq // n_quarters
    n_chunks = b_per_w // BCHUNK
    n_pairs = n_chunks // 2
    mesh = plsc.VectorSubcoreMesh(core_axis_name="c", subcore_axis_name="s")

    @functools.partial(
        pl.kernel,
        mesh=mesh,
        out_type=jax.ShapeDtypeStruct((SEQ, D, Bq), jnp.float32),
        scratch_types=[
            pltpu.VMEM((DGROUP, V), jnp.float32),
            pltpu.VMEM((SEQ, BCHUNK), jnp.int32),
            pltpu.VMEM((SEQ, BCHUNK), jnp.int32),
            pltpu.VMEM((HALF_SEQ, DGROUP, BCHUNK), jnp.float32),
            pltpu.VMEM((HALF_SEQ, DGROUP, BCHUNK), jnp.float32),
            pltpu.SemaphoreType.DMA,
            pltpu.SemaphoreType.DMA,
            pltpu.SemaphoreType.DMA,
            pltpu.SemaphoreType.DMA,
        ],
        compiler_params=pltpu.CompilerParams(needs_layout_passes=False),
    )
    def gather_kernel(
        idxT_hbm, tblT_hbm, out_hbm, tbl_v, idx0, idx1, st0, st1, si0, si1, so0, so1
    ):
        wid = lax.axis_index("s") * info.num_cores + lax.axis_index("c")
        g = wid % n_dgroups
        q = wid // n_dgroups
        b_base = q * b_per_w
        idxb = (idx0, idx1)
        si = (si0, si1)
        st = (st0, st1)
        so = (so0, so1)

        pltpu.sync_copy(tblT_hbm.at[pl.ds(g * DGROUP, DGROUP), :], tbl_v)

        d_splats = [jnp.full((LANES,), d, jnp.int32) for d in range(DGROUP)]

        def fire_idx(c, b):
            pltpu.async_copy(
                idxT_hbm.at[:, pl.ds(pl.multiple_of(b_base + c * BCHUNK, 128), BCHUNK)],
                idxb[b],
                si[b],
            )

        def wait_idx(b):
            pltpu.make_async_copy(
                idxT_hbm.at[:, pl.ds(0, BCHUNK)], idxb[b], si[b]
            ).wait()

        def compute_half(ib, h, c):
            @plsc.parallel_loop(0, HALF_SEQ, unroll=2)
            def s_body(si_):
                s = h * HALF_SEQ + si_
                for bg in range(BCHUNK // LANES):
                    iv = idxb[ib][s, pl.ds(bg * LANES, LANES)]
                    for d in range(DGROUP):
                        v = plsc.load_gather(tbl_v, [d_splats[d], iv])
                        st[h][si_, d, pl.ds(bg * LANES, LANES)] = v

        def start_out(h, c):
            pltpu.async_copy(
                st[h],
                out_hbm.at[
                    pl.ds(h * HALF_SEQ, HALF_SEQ),
                    pl.ds(g * DGROUP, DGROUP),
                    pl.ds(pl.multiple_of(b_base + c * BCHUNK, 128), BCHUNK),
                ],
                so[h],
            )

        def wait_out(h):
            pltpu.make_async_copy(
                st[h],
                out_hbm.at[
                    pl.ds(0, HALF_SEQ), pl.ds(0, DGROUP), pl.ds(0, BCHUNK)
                ],
                so[h],
            ).wait()

        def do_chunk(ib, c):
            for h in range(2):
                @pl.when(c > 0)
                def _():
                    wait_out(h)

                compute_half(ib, h, c)
                start_out(h, c)

        fire_idx(0, 0)

        def pair_body(p, carry):
            c = 2 * p
            wait_idx(0)
            fire_idx(c + 1, 1)
            do_chunk(0, c)
            wait_idx(1)

            @pl.when(p + 1 < n_pairs)
            def _():
                fire_idx(c + 2, 0)

            do_chunk(1, c + 1)
            return carry

        lax.fori_loop(0, n_pairs, pair_body, 0)
        wait_out(0)
        wait_out(1)

    return gather_kernel


def kernel(dayssinceepoch, pos_embed):
    Bq, S = dayssinceepoch.shape
    V, D = pos_embed.shape
    idxT = dayssinceepoch.astype(jnp.int32).T
    tblT = pos_embed.T
    outT = _make_sc_gather(Bq, V, D)(idxT, tblT)
    return jnp.transpose(outT, (2, 0, 1))


# speedup vs baseline: 1.2401x; 1.2401x over previous
import functools

import jax
import jax.numpy as jnp
from jax import lax
from jax.experimental import pallas as pl
from jax.experimental.pallas import tpu as pltpu
from jax.experimental.pallas import tpu_sc as plsc

SEQ = 50
HALF_SEQ = 25
LANES = 16
BCHUNK = 128
DGROUP = 8


def _make_sc_gather(Bq, V, D):
    info = plsc.get_sparse_core_info()
    num_workers = info.num_cores * info.num_subcores
    n_dgroups = D // DGROUP
    n_quarters = num_workers // n_dgroups
    b_per_w = Bq // n_quarters
    n_chunks = b_per_w // BCHUNK
    n_pairs = n_chunks // 2
    mesh = plsc.VectorSubcoreMesh(core_axis_name="c", subcore_axis_name="s")

    @functools.partial(
        pl.kernel,
        mesh=mesh,
        out_type=jax.ShapeDtypeStruct((SEQ, D, Bq), jnp.float32),
        scratch_types=[
            pltpu.VMEM((DGROUP, V), jnp.float32),
            pltpu.VMEM((SEQ, BCHUNK), jnp.int32),
            pltpu.VMEM((SEQ, BCHUNK), jnp.int32),
            pltpu.VMEM((HALF_SEQ, DGROUP, BCHUNK), jnp.float32),
            pltpu.VMEM((HALF_SEQ, DGROUP, BCHUNK), jnp.float32),
            pltpu.SemaphoreType.DMA,
            pltpu.SemaphoreType.DMA,
            pltpu.SemaphoreType.DMA,
            pltpu.SemaphoreType.DMA,
        ],
        compiler_params=pltpu.CompilerParams(needs_layout_passes=False),
    )
    def gather_kernel(
        idxT_hbm, tblT_hbm, out_hbm, tbl_v, idx0, idx1, st0, st1, si0, si1, so0, so1
    ):
        wid = lax.axis_index("s") * info.num_cores + lax.axis_index("c")
        g = wid % n_dgroups
        q = wid // n_dgroups
        b_base = q * b_per_w
        idxb = (idx0, idx1)
        si = (si0, si1)
        st = (st0, st1)
        so = (so0, so1)

        pltpu.sync_copy(tblT_hbm.at[pl.ds(g * DGROUP, DGROUP), :], tbl_v)

        d_splats = [jnp.full((LANES,), d, jnp.int32) for d in range(DGROUP)]

        def fire_idx(c, b):
            pltpu.async_copy(
                idxT_hbm.at[:, pl.ds(pl.multiple_of(b_base + c * BCHUNK, 128), BCHUNK)],
                idxb[b],
                si[b],
            )

        def wait_idx(b):
            pltpu.make_async_copy(
                idxT_hbm.at[:, pl.ds(0, BCHUNK)], idxb[b], si[b]
            ).wait()

        def compute_half(ib, h, c):
            @plsc.parallel_loop(0, HALF_SEQ)
            def s_body(si_):
                s = h * HALF_SEQ + si_
                for bg in range(BCHUNK // LANES):
                    iv = idxb[ib][s, pl.ds(bg * LANES, LANES)]
                    for d in range(DGROUP):
                        v = plsc.load_gather(tbl_v, [d_splats[d], iv])
                        st[h][si_, d, pl.ds(bg * LANES, LANES)] = v

        def start_out(h, c):
            pltpu.async_copy(
                st[h],
                out_hbm.at[
                    pl.ds(h * HALF_SEQ, HALF_SEQ),
                    pl.ds(g * DGROUP, DGROUP),
                    pl.ds(pl.multiple_of(b_base + c * BCHUNK, 128), BCHUNK),
                ],
                so[h],
            )

        def wait_out(h):
            pltpu.make_async_copy(
                st[h],
                out_hbm.at[
                    pl.ds(0, HALF_SEQ), pl.ds(0, DGROUP), pl.ds(0, BCHUNK)
                ],
                so[h],
            ).wait()

        def do_chunk(ib, c):
            for h in range(2):
                @pl.when(c > 0)
                def _():
                    wait_out(h)

                compute_half(ib, h, c)
                start_out(h, c)

        fire_idx(0, 0)

        def pair_body(p, carry):
            c = 2 * p
            wait_idx(0)
            fire_idx(c + 1, 1)
            do_chunk(0, c)
            wait_idx(1)

            @pl.when(p + 1 < n_pairs)
            def _():
                fire_idx(c + 2, 0)

            do_chunk(1, c + 1)
            return carry

        lax.fori_loop(0, n_pairs, pair_body, 0)
        wait_out(0)
        wait_out(1)

    return gather_kernel


def kernel(dayssinceepoch, pos_embed):
    Bq, S = dayssinceepoch.shape
    V, D = pos_embed.shape
    idxT = dayssinceepoch.astype(jnp.int32).T
    tblT = pos_embed.T
    outT = _make_sc_gather(Bq, V, D)(idxT, tblT)
    return jnp.transpose(outT, (2, 0, 1))


# speedup vs baseline: 1.8608x; 1.5005x over previous
import functools

import jax
import jax.numpy as jnp
from jax import lax
from jax.experimental import pallas as pl
from jax.experimental.pallas import tpu as pltpu
from jax.experimental.pallas import tpu_sc as plsc

SEQ = 50
HALF_SEQ = 25
LANES = 16
BCHUNK = 128
DGROUP = 8


def _make_sc_gather(Bq, V, D):
    info = plsc.get_sparse_core_info()
    num_workers = info.num_cores * info.num_subcores
    n_dgroups = D // DGROUP
    n_quarters = num_workers // n_dgroups
    b_per_w = Bq // n_quarters
    n_chunks = b_per_w // BCHUNK
    n_pairs = n_chunks // 2
    mesh = plsc.VectorSubcoreMesh(core_axis_name="c", subcore_axis_name="s")

    @functools.partial(
        pl.kernel,
        mesh=mesh,
        out_type=jax.ShapeDtypeStruct((SEQ, D, Bq), jnp.float32),
        scratch_types=[
            pltpu.VMEM((DGROUP, V), jnp.float32),
            pltpu.VMEM((SEQ, BCHUNK), jnp.int32),
            pltpu.VMEM((SEQ, BCHUNK), jnp.int32),
            pltpu.VMEM((HALF_SEQ, DGROUP, BCHUNK), jnp.float32),
            pltpu.VMEM((HALF_SEQ, DGROUP, BCHUNK), jnp.float32),
            pltpu.SemaphoreType.DMA,
            pltpu.SemaphoreType.DMA,
            pltpu.SemaphoreType.DMA,
            pltpu.SemaphoreType.DMA,
        ],
        compiler_params=pltpu.CompilerParams(needs_layout_passes=False),
    )
    def gather_kernel(
        idxT_hbm, tblT_hbm, out_hbm, tbl_v, idx0, idx1, st0, st1, si0, si1, so0, so1
    ):
        wid = lax.axis_index("s") * info.num_cores + lax.axis_index("c")
        g = wid % n_dgroups
        q = wid // n_dgroups
        b_base = q * b_per_w
        idxb = (idx0, idx1)
        si = (si0, si1)
        st = (st0, st1)
        so = (so0, so1)

        pltpu.sync_copy(tblT_hbm.at[pl.ds(g * DGROUP, DGROUP), :], tbl_v)

        d_splats = [jnp.full((LANES,), d, jnp.int32) for d in range(DGROUP)]

        def fire_idx(c, b):
            pltpu.async_copy(
                idxT_hbm.at[:, pl.ds(pl.multiple_of(b_base + c * BCHUNK, 128), BCHUNK)],
                idxb[b],
                si[b],
            )

        def wait_idx(b):
            pltpu.make_async_copy(
                idxT_hbm.at[:, pl.ds(0, BCHUNK)], idxb[b], si[b]
            ).wait()

        def compute_half(ib, h, c):
            @plsc.parallel_loop(0, HALF_SEQ * (BCHUNK // LANES))
            def s_body(n):
                si_ = n // (BCHUNK // LANES)
                bg = n % (BCHUNK // LANES)
                s = h * HALF_SEQ + si_
                iv = idxb[ib][s, pl.ds(bg * LANES, LANES)]
                for d in range(DGROUP):
                    v = plsc.load_gather(tbl_v, [d_splats[d], iv])
                    st[h][si_, d, pl.ds(bg * LANES, LANES)] = v

        def start_out(h, c):
            pltpu.async_copy(
                st[h],
                out_hbm.at[
                    pl.ds(h * HALF_SEQ, HALF_SEQ),
                    pl.ds(g * DGROUP, DGROUP),
                    pl.ds(pl.multiple_of(b_base + c * BCHUNK, 128), BCHUNK),
                ],
                so[h],
            )

        def wait_out(h):
            pltpu.make_async_copy(
                st[h],
                out_hbm.at[
                    pl.ds(0, HALF_SEQ), pl.ds(0, DGROUP), pl.ds(0, BCHUNK)
                ],
                so[h],
            ).wait()

        def do_chunk(ib, c):
            for h in range(2):
                @pl.when(c > 0)
                def _():
                    wait_out(h)

                compute_half(ib, h, c)
                start_out(h, c)

        fire_idx(0, 0)

        def pair_body(p, carry):
            c = 2 * p
            wait_idx(0)
            fire_idx(c + 1, 1)
            do_chunk(0, c)
            wait_idx(1)

            @pl.when(p + 1 < n_pairs)
            def _():
                fire_idx(c + 2, 0)

            do_chunk(1, c + 1)
            return carry

        lax.fori_loop(0, n_pairs, pair_body, 0)
        wait_out(0)
        wait_out(1)

    return gather_kernel


def kernel(dayssinceepoch, pos_embed):
    Bq, S = dayssinceepoch.shape
    V, D = pos_embed.shape
    idxT = dayssinceepoch.astype(jnp.int32).T
    tblT = pos_embed.T
    outT = _make_sc_gather(Bq, V, D)(idxT, tblT)
    return jnp.transpose(outT, (2, 0, 1))


# speedup vs baseline: 1.8720x; 1.0060x over previous
import functools

import jax
import jax.numpy as jnp
from jax import lax
from jax.experimental import pallas as pl
from jax.experimental.pallas import tpu as pltpu
from jax.experimental.pallas import tpu_sc as plsc

SEQ = 50
HALF_SEQ = 25
LANES = 16
BCHUNK = 128
DGROUP = 8


def _make_sc_gather(Bq, V, D):
    info = plsc.get_sparse_core_info()
    num_workers = info.num_cores * info.num_subcores
    n_dgroups = D // DGROUP
    n_quarters = num_workers // n_dgroups
    b_per_w = Bq // n_quarters
    n_chunks = b_per_w // BCHUNK
    n_pairs = n_chunks // 2
    mesh = plsc.VectorSubcoreMesh(core_axis_name="c", subcore_axis_name="s")

    @functools.partial(
        pl.kernel,
        mesh=mesh,
        out_type=jax.ShapeDtypeStruct((SEQ, D, Bq), jnp.float32),
        scratch_types=[
            pltpu.VMEM((DGROUP, V), jnp.float32),
            pltpu.VMEM((SEQ, BCHUNK), jnp.int32),
            pltpu.VMEM((SEQ, BCHUNK), jnp.int32),
            pltpu.VMEM((HALF_SEQ, DGROUP, BCHUNK), jnp.float32),
            pltpu.VMEM((HALF_SEQ, DGROUP, BCHUNK), jnp.float32),
            pltpu.SemaphoreType.DMA,
            pltpu.SemaphoreType.DMA,
            pltpu.SemaphoreType.DMA,
            pltpu.SemaphoreType.DMA,
        ],
        compiler_params=pltpu.CompilerParams(needs_layout_passes=False),
    )
    def gather_kernel(
        idxT_hbm, tblT_hbm, out_hbm, tbl_v, idx0, idx1, st0, st1, si0, si1, so0, so1
    ):
        wid = lax.axis_index("s") * info.num_cores + lax.axis_index("c")
        g = wid % n_dgroups
        q = wid // n_dgroups
        b_base = q * b_per_w
        idxb = (idx0, idx1)
        si = (si0, si1)
        st = (st0, st1)
        so = (so0, so1)

        pltpu.sync_copy(tblT_hbm.at[pl.ds(g * DGROUP, DGROUP), :], tbl_v)

        d_splats = [jnp.full((LANES,), d, jnp.int32) for d in range(DGROUP)]

        def fire_idx(c, b):
            pltpu.async_copy(
                idxT_hbm.at[:, pl.ds(pl.multiple_of(b_base + c * BCHUNK, 128), BCHUNK)],
                idxb[b],
                si[b],
            )

        def wait_idx(b):
            pltpu.make_async_copy(
                idxT_hbm.at[:, pl.ds(0, BCHUNK)], idxb[b], si[b]
            ).wait()

        def compute_half(ib, h, c):
            @plsc.parallel_loop(0, HALF_SEQ * (BCHUNK // LANES), unroll=2)
            def s_body(n):
                si_ = n // (BCHUNK // LANES)
                bg = n % (BCHUNK // LANES)
                s = h * HALF_SEQ + si_
                iv = idxb[ib][s, pl.ds(bg * LANES, LANES)]
                for d in range(DGROUP):
                    v = plsc.load_gather(tbl_v, [d_splats[d], iv])
                    st[h][si_, d, pl.ds(bg * LANES, LANES)] = v

        def start_out(h, c):
            pltpu.async_copy(
                st[h],
                out_hbm.at[
                    pl.ds(h * HALF_SEQ, HALF_SEQ),
                    pl.ds(g * DGROUP, DGROUP),
                    pl.ds(pl.multiple_of(b_base + c * BCHUNK, 128), BCHUNK),
                ],
                so[h],
            )

        def wait_out(h):
            pltpu.make_async_copy(
                st[h],
                out_hbm.at[
                    pl.ds(0, HALF_SEQ), pl.ds(0, DGROUP), pl.ds(0, BCHUNK)
                ],
                so[h],
            ).wait()

        def do_chunk(ib, c):
            for h in range(2):
                @pl.when(c > 0)
                def _():
                    wait_out(h)

                compute_half(ib, h, c)
                start_out(h, c)

        fire_idx(0, 0)

        def pair_body(p, carry):
            c = 2 * p
            wait_idx(0)
            fire_idx(c + 1, 1)
            do_chunk(0, c)
            wait_idx(1)

            @pl.when(p + 1 < n_pairs)
            def _():
                fire_idx(c + 2, 0)

            do_chunk(1, c + 1)
            return carry

        lax.fori_loop(0, n_pairs, pair_body, 0)
        wait_out(0)
        wait_out(1)

    return gather_kernel


def kernel(dayssinceepoch, pos_embed):
    Bq, S = dayssinceepoch.shape
    V, D = pos_embed.shape
    idxT = dayssinceepoch.astype(jnp.int32).T
    tblT = pos_embed.T
    outT = _make_sc_gather(Bq, V, D)(idxT, tblT)
    return jnp.transpose(outT, (2, 0, 1))
